# baseline (speedup 1.0000x reference)
import functools

import jax
import jax.numpy as jnp
from jax import lax
from jax.experimental import pallas as pl
from jax.experimental.pallas import tpu as pltpu
from jax.experimental.pallas import tpu_sc as plsc
from jax.experimental import layout as jex_layout

NC, NS = 2, 16
NW = NC * NS
CH = 128
GROUP = 1024


def _sc_gather(cidx_eo, xidx_eo, nidx_eo, cemb, xemb, B, K, D):
    BK = B * K
    hpw = B // 2 // NW
    nhpw = BK // 2 // NW
    GH = GROUP // 2
    mesh = plsc.VectorSubcoreMesh(core_axis_name="c", subcore_axis_name="s")

    @functools.partial(
        pl.kernel,
        out_type=(
            jax.ShapeDtypeStruct((B // 2, 2 * D), jnp.float32),
            jax.ShapeDtypeStruct((B // 2, 2 * D), jnp.float32),
            jax.ShapeDtypeStruct((K, B // 2, 2 * D), jnp.float32),
        ),
        mesh=mesh,
        compiler_params=pltpu.CompilerParams(use_tc_tiling_on_sc=False),
        scratch_types=[
            pltpu.VMEM((2, hpw), jnp.int32),
            pltpu.VMEM((2, hpw), jnp.int32),
            pltpu.VMEM((2, nhpw), jnp.int32),
            pltpu.VMEM((GH, D), jnp.float32),
            pltpu.VMEM((GH, D), jnp.float32),
            pltpu.SemaphoreType.DMA,
        ],
    )
    def gather_kernel(cidx_h, xidx_h, nidx_h, cemb_h, xemb_h,
                      ce_pk_o, ct_pk_o, ne_pk_o, idx_c, idx_x, idx_n,
                      rows_e, rows_o, sem):
        wid = lax.axis_index("s") * NC + lax.axis_index("c")
        pltpu.sync_copy(cidx_h.at[:, pl.ds(wid * hpw, hpw)], idx_c)
        pltpu.sync_copy(xidx_h.at[:, pl.ds(wid * hpw, hpw)], idx_x)
        pltpu.sync_copy(nidx_h.at[:, pl.ds(wid * nhpw, nhpw)], idx_n)

        def group(table_h, idx_ref, idx_off, dst, nh):
            cps = []
            for half, buf in ((0, rows_e), (1, rows_o)):
                for c in range(nh // CH):
                    cps.append(pltpu.async_copy(
                        table_h.at[idx_ref.at[half,
                                              pl.ds(idx_off + c * CH, CH)]],
                        buf.at[pl.ds(c * CH, CH)], sem))
            for cp in cps:
                cp.wait()
            pltpu.sync_copy(rows_e.at[pl.ds(0, nh), :],
                            dst.at[:, pl.ds(0, D)])
            pltpu.sync_copy(rows_o.at[pl.ds(0, nh), :],
                            dst.at[:, pl.ds(D, D)])

        group(cemb_h, idx_c, 0,
              ce_pk_o.at[pl.ds(wid * hpw, hpw), :], hpw)
        group(xemb_h, idx_x, 0,
              ct_pk_o.at[pl.ds(wid * hpw, hpw), :], hpw)
        for g in range(nhpw // GH):
            half_row = wid * nhpw + g * GH
            k = half_row // (B // 2)
            j0 = half_row % (B // 2)
            group(xemb_h, idx_n, g * GH,
                  ne_pk_o.at[k].at[pl.ds(j0, GH), :], GH)

    return gather_kernel(cidx_eo, xidx_eo, nidx_eo, cemb, xemb)


def _softplus(x):
    return jnp.maximum(x, 0.0) + jnp.log1p(jnp.exp(-jnp.abs(x)))


def _half_mask(rows, cols):
    r = lax.broadcasted_iota(jnp.int32, (rows, cols), 0)
    c = lax.broadcasted_iota(jnp.int32, (rows, cols), 1)
    return jnp.where(r // 64 == c, 1.0, 0.0).astype(jnp.float32)


def _tc_mlp_pos(ce_pk, ct_pk, W1d, b1d, W2d, b2d, B, D, H):
    BLK = 512
    npk = B // 2
    nblk = npk // BLK

    def body(ce_ref, ct_ref, w1_ref, b1_ref, w2_ref, b2_ref,
             ce2_ref, pos_ref):
        i = pl.program_id(0)
        h = jnp.dot(ce_ref[...], w1_ref[...],
                    preferred_element_type=jnp.float32) + b1_ref[...]
        h = jnp.maximum(h, 0.0)
        ce2 = jnp.dot(h, w2_ref[...],
                      preferred_element_type=jnp.float32) + b2_ref[...]
        ce2_ref[...] = ce2
        prod = ce2 * ct_ref[...]
        pos = jnp.dot(prod, _half_mask(2 * D, 2),
                      preferred_element_type=jnp.float32)
        part = jnp.sum(_softplus(-pos)) * (1.0 / B)

        @pl.when(i == 0)
        def _():
            pos_ref[0, 0] = part

        @pl.when(i != 0)
        def _():
            pos_ref[0, 0] += part

    return pl.pallas_call(
        body,
        grid=(nblk,),
        in_specs=[
            pl.BlockSpec((BLK, 2 * D), lambda i: (i, 0)),
            pl.BlockSpec((BLK, 2 * D), lambda i: (i, 0)),
            pl.BlockSpec((2 * D, 2 * H), lambda i: (0, 0)),
            pl.BlockSpec((1, 2 * H), lambda i: (0, 0)),
            pl.BlockSpec((2 * H, 2 * D), lambda i: (0, 0)),
            pl.BlockSpec((1, 2 * D), lambda i: (0, 0)),
        ],
        out_specs=[
            pl.BlockSpec((BLK, 2 * D), lambda i: (i, 0)),
            pl.BlockSpec(memory_space=pltpu.SMEM),
        ],
        out_shape=[
            jax.ShapeDtypeStruct((npk, 2 * D), jnp.float32),
            jax.ShapeDtypeStruct((1, 1), jnp.float32),
        ],
    )(ce_pk, ct_pk, W1d, b1d, W2d, b2d)


def _tc_neg(neg_pk, ce2_pk, B, K, D):
    BLK = 512
    npk = B // 2
    nblk = npk // BLK

    def body(ne_ref, ce2_ref, out_ref):
        i = pl.program_id(0)
        ce2 = ce2_ref[...]
        prods = [ne_ref[k] * ce2 for k in range(K)]
        pall = jnp.concatenate(prods, axis=1)
        scores = jnp.dot(pall, _half_mask(K * 2 * D, 2 * K),
                         preferred_element_type=jnp.float32)
        part = jnp.sum(_softplus(scores)) * (1.0 / (B * K))

        @pl.when(i == 0)
        def _():
            out_ref[0, 0] = part

        @pl.when(i != 0)
        def _():
            out_ref[0, 0] += part

    return pl.pallas_call(
        body,
        grid=(nblk,),
        in_specs=[
            pl.BlockSpec((K, BLK, 2 * D), lambda i: (0, i, 0)),
            pl.BlockSpec((BLK, 2 * D), lambda i: (i, 0)),
        ],
        out_specs=pl.BlockSpec(memory_space=pltpu.SMEM),
        out_shape=jax.ShapeDtypeStruct((1, 1), jnp.float32),
    )(neg_pk, ce2_pk)


def _blockdiag2(W):
    n, m = W.shape
    z = jnp.zeros((n, m), jnp.float32)
    return jnp.concatenate([
        jnp.concatenate([W, z], axis=1),
        jnp.concatenate([z, W], axis=1),
    ], axis=0)


def kernel(center_word_indices, context_word_indices, negative_word_indices,
           center_emb, context_emb, W1, b1, W2, b2):
    B, K = negative_word_indices.shape
    V, D = center_emb.shape
    H = W1.shape[1]
    cidx = center_word_indices.astype(jnp.int32)
    xidx = context_word_indices.astype(jnp.int32)
    nidx = negative_word_indices.astype(jnp.int32).T.reshape(-1)

    def eo(a):
        return jnp.stack([a[0::2], a[1::2]])

    lin = jex_layout.Layout(major_to_minor=(0,), tiling=((1024,),))
    cemb_lin = jex_layout.with_layout_constraint(
        center_emb.reshape(-1), lin).reshape(V, D)
    xemb_lin = jex_layout.with_layout_constraint(
        context_emb.reshape(-1), lin).reshape(V, D)
    ce_pk, ct_pk, neg_pk = _sc_gather(eo(cidx), eo(xidx), eo(nidx),
                                      cemb_lin, xemb_lin, B, K, D)
    W1d = _blockdiag2(W1)
    W2d = _blockdiag2(W2)
    b1d = jnp.concatenate([b1, b1]).reshape(1, 2 * H)
    b2d = jnp.concatenate([b2, b2]).reshape(1, 2 * D)
    ce2_pk, pos_loss = _tc_mlp_pos(ce_pk, ct_pk, W1d, b1d, W2d, b2d, B, D, H)
    neg_loss = _tc_neg(neg_pk, ce2_pk, B, K, D)
    return pos_loss[0, 0] + neg_loss[0, 0]


# speedup vs baseline: 1.0009x; 1.0009x over previous
import functools

import jax
import jax.numpy as jnp
from jax import lax
from jax.experimental import pallas as pl
from jax.experimental.pallas import tpu as pltpu
from jax.experimental.pallas import tpu_sc as plsc
from jax.experimental import layout as jex_layout

NC, NS = 2, 16
NW = NC * NS
CH = 128
GROUP = 512


def _sc_gather(cidx_eo, xidx_eo, nidx_eo, cemb, xemb, B, K, D):
    BK = B * K
    Dp = 2 * D
    hpw = B // 2 // NW
    nhpw = BK // 2 // NW
    GH = GROUP // 2
    mesh = plsc.VectorSubcoreMesh(core_axis_name="c", subcore_axis_name="s")

    @functools.partial(
        pl.kernel,
        out_type=(
            jax.ShapeDtypeStruct((B // 2, 2 * D), jnp.float32),
            jax.ShapeDtypeStruct((B // 2, 2 * D), jnp.float32),
            jax.ShapeDtypeStruct((K, B // 2, 2 * D), jnp.float32),
        ),
        mesh=mesh,
        compiler_params=pltpu.CompilerParams(use_tc_tiling_on_sc=False),
        scratch_types=[
            pltpu.VMEM((2, hpw), jnp.int32),
            pltpu.VMEM((2, hpw), jnp.int32),
            pltpu.VMEM((2, nhpw), jnp.int32),
            pltpu.VMEM((GH, Dp), jnp.float32),
            pltpu.VMEM((GH, Dp), jnp.float32),
            pltpu.SemaphoreType.DMA,
        ],
    )
    def gather_kernel(cidx_h, xidx_h, nidx_h, cemb_h, xemb_h,
                      ce_pk_o, ct_pk_o, ne_pk_o, idx_c, idx_x, idx_n,
                      rows_e, rows_o, sem):
        wid = lax.axis_index("s") * NC + lax.axis_index("c")
        pltpu.sync_copy(cidx_h.at[:, pl.ds(wid * hpw, hpw)], idx_c)
        pltpu.sync_copy(xidx_h.at[:, pl.ds(wid * hpw, hpw)], idx_x)
        pltpu.sync_copy(nidx_h.at[:, pl.ds(wid * nhpw, nhpw)], idx_n)

        def group(table_h, idx_ref, idx_off, dst, nh):
            cps = []
            for half, buf in ((0, rows_e), (1, rows_o)):
                for c in range(nh // CH):
                    cps.append(pltpu.async_copy(
                        table_h.at[idx_ref.at[half,
                                              pl.ds(idx_off + c * CH, CH)]],
                        buf.at[pl.ds(c * CH, CH)], sem))
            for cp in cps:
                cp.wait()
            pltpu.sync_copy(rows_e.at[pl.ds(0, nh), pl.ds(0, D)],
                            dst.at[:, pl.ds(0, D)])
            pltpu.sync_copy(rows_o.at[pl.ds(0, nh), pl.ds(0, D)],
                            dst.at[:, pl.ds(D, D)])

        group(cemb_h, idx_c, 0,
              ce_pk_o.at[pl.ds(wid * hpw, hpw), :], hpw)
        group(xemb_h, idx_x, 0,
              ct_pk_o.at[pl.ds(wid * hpw, hpw), :], hpw)
        for g in range(nhpw // GH):
            half_row = wid * nhpw + g * GH
            k = half_row // (B // 2)
            j0 = half_row % (B // 2)
            group(xemb_h, idx_n, g * GH,
                  ne_pk_o.at[k].at[pl.ds(j0, GH), :], GH)

    return gather_kernel(cidx_eo, xidx_eo, nidx_eo, cemb, xemb)


def _softplus(x):
    return jnp.maximum(x, 0.0) + jnp.log1p(jnp.exp(-jnp.abs(x)))


def _half_mask(rows, cols):
    r = lax.broadcasted_iota(jnp.int32, (rows, cols), 0)
    c = lax.broadcasted_iota(jnp.int32, (rows, cols), 1)
    return jnp.where(r // 64 == c, 1.0, 0.0).astype(jnp.float32)


def _tc_mlp_pos(ce_pk, ct_pk, W1d, b1d, W2d, b2d, B, D, H):
    BLK = 512
    npk = B // 2
    nblk = npk // BLK

    def body(ce_ref, ct_ref, w1_ref, b1_ref, w2_ref, b2_ref,
             ce2_ref, pos_ref):
        i = pl.program_id(0)
        h = jnp.dot(ce_ref[...], w1_ref[...],
                    preferred_element_type=jnp.float32) + b1_ref[...]
        h = jnp.maximum(h, 0.0)
        ce2 = jnp.dot(h, w2_ref[...],
                      preferred_element_type=jnp.float32) + b2_ref[...]
        ce2_ref[...] = ce2
        prod = ce2 * ct_ref[...]
        pos = jnp.dot(prod, _half_mask(2 * D, 2),
                      preferred_element_type=jnp.float32)
        part = jnp.sum(_softplus(-pos)) * (1.0 / B)

        @pl.when(i == 0)
        def _():
            pos_ref[0, 0] = part

        @pl.when(i != 0)
        def _():
            pos_ref[0, 0] += part

    return pl.pallas_call(
        body,
        grid=(nblk,),
        in_specs=[
            pl.BlockSpec((BLK, 2 * D), lambda i: (i, 0)),
            pl.BlockSpec((BLK, 2 * D), lambda i: (i, 0)),
            pl.BlockSpec((2 * D, 2 * H), lambda i: (0, 0)),
            pl.BlockSpec((1, 2 * H), lambda i: (0, 0)),
            pl.BlockSpec((2 * H, 2 * D), lambda i: (0, 0)),
            pl.BlockSpec((1, 2 * D), lambda i: (0, 0)),
        ],
        out_specs=[
            pl.BlockSpec((BLK, 2 * D), lambda i: (i, 0)),
            pl.BlockSpec(memory_space=pltpu.SMEM),
        ],
        out_shape=[
            jax.ShapeDtypeStruct((npk, 2 * D), jnp.float32),
            jax.ShapeDtypeStruct((1, 1), jnp.float32),
        ],
    )(ce_pk, ct_pk, W1d, b1d, W2d, b2d)


def _tc_neg(neg_pk, ce2_pk, B, K, D):
    BLK = 512
    npk = B // 2
    nblk = npk // BLK

    def body(ne_ref, ce2_ref, out_ref):
        i = pl.program_id(0)
        ce2 = ce2_ref[...]
        prods = [ne_ref[k] * ce2 for k in range(K)]
        pall = jnp.concatenate(prods, axis=1)
        scores = jnp.dot(pall, _half_mask(K * 2 * D, 2 * K),
                         preferred_element_type=jnp.float32)
        part = jnp.sum(_softplus(scores)) * (1.0 / (B * K))

        @pl.when(i == 0)
        def _():
            out_ref[0, 0] = part

        @pl.when(i != 0)
        def _():
            out_ref[0, 0] += part

    return pl.pallas_call(
        body,
        grid=(nblk,),
        in_specs=[
            pl.BlockSpec((K, BLK, 2 * D), lambda i: (0, i, 0)),
            pl.BlockSpec((BLK, 2 * D), lambda i: (i, 0)),
        ],
        out_specs=pl.BlockSpec(memory_space=pltpu.SMEM),
        out_shape=jax.ShapeDtypeStruct((1, 1), jnp.float32),
    )(neg_pk, ce2_pk)


def _blockdiag2(W):
    n, m = W.shape
    z = jnp.zeros((n, m), jnp.float32)
    return jnp.concatenate([
        jnp.concatenate([W, z], axis=1),
        jnp.concatenate([z, W], axis=1),
    ], axis=0)


def kernel(center_word_indices, context_word_indices, negative_word_indices,
           center_emb, context_emb, W1, b1, W2, b2):
    B, K = negative_word_indices.shape
    V, D = center_emb.shape
    H = W1.shape[1]
    cidx = center_word_indices.astype(jnp.int32)
    xidx = context_word_indices.astype(jnp.int32)
    nidx = negative_word_indices.astype(jnp.int32).T.reshape(-1)

    def eo(a):
        return jnp.stack([a[0::2], a[1::2]])

    cemb_p = jnp.pad(center_emb, ((0, 0), (0, D)))
    xemb_p = jnp.pad(context_emb, ((0, 0), (0, D)))
    ce_pk, ct_pk, neg_pk = _sc_gather(eo(cidx), eo(xidx), eo(nidx),
                                      cemb_p, xemb_p, B, K, D)
    W1d = _blockdiag2(W1)
    W2d = _blockdiag2(W2)
    b1d = jnp.concatenate([b1, b1]).reshape(1, 2 * H)
    b2d = jnp.concatenate([b2, b2]).reshape(1, 2 * D)
    ce2_pk, pos_loss = _tc_mlp_pos(ce_pk, ct_pk, W1d, b1d, W2d, b2d, B, D, H)
    neg_loss = _tc_neg(neg_pk, ce2_pk, B, K, D)
    return pos_loss[0, 0] + neg_loss[0, 0]


# speedup vs baseline: 1.5255x; 1.5241x over previous
import functools

import jax
import jax.numpy as jnp
from jax import lax
from jax.experimental import pallas as pl
from jax.experimental.pallas import tpu as pltpu
from jax.experimental.pallas import tpu_sc as plsc
from jax.experimental import layout as jex_layout

NC, NS = 2, 16
NW = NC * NS
CH = 128
GROUP = 512


def _sc_gather(cidx_eo, xidx_eo, nidx_eo, cemb, xemb, B, K, D):
    BK = B * K
    Dp = 2 * D
    hpw = B // 2 // NW
    nhpw = BK // 2 // NW
    GH = GROUP // 2
    mesh = plsc.VectorSubcoreMesh(core_axis_name="c", subcore_axis_name="s")

    @functools.partial(
        pl.kernel,
        out_type=(
            jax.ShapeDtypeStruct((B // 2, 2 * D), jnp.float32),
            jax.ShapeDtypeStruct((B // 2, 2 * D), jnp.float32),
            jax.ShapeDtypeStruct((K, B // 2, 2 * D), jnp.float32),
        ),
        mesh=mesh,
        compiler_params=pltpu.CompilerParams(use_tc_tiling_on_sc=False),
        scratch_types=[
            pltpu.VMEM((2, hpw), jnp.int32),
            pltpu.VMEM((2, hpw), jnp.int32),
            pltpu.VMEM((2, nhpw), jnp.int32),
            pltpu.VMEM((GH, Dp), jnp.float32),
            pltpu.VMEM((GH, Dp), jnp.float32),
            pltpu.SemaphoreType.DMA,
        ],
    )
    def gather_kernel(cidx_h, xidx_h, nidx_h, cemb_h, xemb_h,
                      ce_pk_o, ct_pk_o, ne_pk_o, idx_c, idx_x, idx_n,
                      rows_e, rows_o, sem):
        wid = lax.axis_index("s") * NC + lax.axis_index("c")
        pltpu.sync_copy(cidx_h.at[:, pl.ds(wid * hpw, hpw)], idx_c)
        pltpu.sync_copy(xidx_h.at[:, pl.ds(wid * hpw, hpw)], idx_x)
        pltpu.sync_copy(nidx_h.at[:, pl.ds(wid * nhpw, nhpw)], idx_n)

        def group(table_h, idx_ref, idx_off, dst, nh):
            cps = []
            for half, buf in ((0, rows_e), (1, rows_o)):
                for c in range(nh // CH):
                    cps.append(pltpu.async_copy(
                        table_h.at[idx_ref.at[half,
                                              pl.ds(idx_off + c * CH, CH)]],
                        buf.at[pl.ds(c * CH, CH)], sem))
            for cp in cps:
                cp.wait()
            pltpu.sync_copy(rows_e.at[pl.ds(0, nh), pl.ds(0, D)],
                            dst.at[:, pl.ds(0, D)])
            pltpu.sync_copy(rows_o.at[pl.ds(0, nh), pl.ds(0, D)],
                            dst.at[:, pl.ds(D, D)])

        group(cemb_h, idx_c, 0,
              ce_pk_o.at[pl.ds(wid * hpw, hpw), :], hpw)
        group(xemb_h, idx_x, 0,
              ct_pk_o.at[pl.ds(wid * hpw, hpw), :], hpw)
        for g in range(nhpw // GH):
            half_row = wid * nhpw + g * GH
            k = half_row // (B // 2)
            j0 = half_row % (B // 2)
            group(xemb_h, idx_n, g * GH,
                  ne_pk_o.at[k].at[pl.ds(j0, GH), :], GH)

    return gather_kernel(cidx_eo, xidx_eo, nidx_eo, cemb, xemb)


def _tc_transpose_pad(tableT, V, D):
    BLKV = 8192
    nblk = V // BLKV

    def body(t_ref, out_ref):
        x = t_ref[...]
        xt = jnp.swapaxes(x, 0, 1)
        out_ref[...] = jnp.concatenate([xt, jnp.zeros_like(xt)], axis=1)

    return pl.pallas_call(
        body,
        grid=(nblk,),
        in_specs=[pl.BlockSpec((D, BLKV), lambda i: (0, i))],
        out_specs=pl.BlockSpec((BLKV, 2 * D), lambda i: (i, 0)),
        out_shape=jax.ShapeDtypeStruct((V, 2 * D), jnp.float32),
    )(tableT)


def _softplus(x):
    return jnp.maximum(x, 0.0) + jnp.log1p(jnp.exp(-jnp.abs(x)))


def _half_mask(rows, cols):
    r = lax.broadcasted_iota(jnp.int32, (rows, cols), 0)
    c = lax.broadcasted_iota(jnp.int32, (rows, cols), 1)
    return jnp.where(r // 64 == c, 1.0, 0.0).astype(jnp.float32)


def _tc_mlp_pos(ce_pk, ct_pk, W1d, b1d, W2d, b2d, B, D, H):
    BLK = 512
    npk = B // 2
    nblk = npk // BLK

    def body(ce_ref, ct_ref, w1_ref, b1_ref, w2_ref, b2_ref,
             ce2_ref, pos_ref):
        i = pl.program_id(0)
        h = jnp.dot(ce_ref[...], w1_ref[...],
                    preferred_element_type=jnp.float32) + b1_ref[...]
        h = jnp.maximum(h, 0.0)
        ce2 = jnp.dot(h, w2_ref[...],
                      preferred_element_type=jnp.float32) + b2_ref[...]
        ce2_ref[...] = ce2
        prod = ce2 * ct_ref[...]
        pos = jnp.dot(prod, _half_mask(2 * D, 2),
                      preferred_element_type=jnp.float32)
        part = jnp.sum(_softplus(-pos)) * (1.0 / B)

        @pl.when(i == 0)
        def _():
            pos_ref[0, 0] = part

        @pl.when(i != 0)
        def _():
            pos_ref[0, 0] += part

    return pl.pallas_call(
        body,
        grid=(nblk,),
        in_specs=[
            pl.BlockSpec((BLK, 2 * D), lambda i: (i, 0)),
            pl.BlockSpec((BLK, 2 * D), lambda i: (i, 0)),
            pl.BlockSpec((2 * D, 2 * H), lambda i: (0, 0)),
            pl.BlockSpec((1, 2 * H), lambda i: (0, 0)),
            pl.BlockSpec((2 * H, 2 * D), lambda i: (0, 0)),
            pl.BlockSpec((1, 2 * D), lambda i: (0, 0)),
        ],
        out_specs=[
            pl.BlockSpec((BLK, 2 * D), lambda i: (i, 0)),
            pl.BlockSpec(memory_space=pltpu.SMEM),
        ],
        out_shape=[
            jax.ShapeDtypeStruct((npk, 2 * D), jnp.float32),
            jax.ShapeDtypeStruct((1, 1), jnp.float32),
        ],
    )(ce_pk, ct_pk, W1d, b1d, W2d, b2d)


def _tc_neg(neg_pk, ce2_pk, B, K, D):
    BLK = 512
    npk = B // 2
    nblk = npk // BLK

    def body(ne_ref, ce2_ref, out_ref):
        i = pl.program_id(0)
        ce2 = ce2_ref[...]
        prods = [ne_ref[k] * ce2 for k in range(K)]
        pall = jnp.concatenate(prods, axis=1)
        scores = jnp.dot(pall, _half_mask(K * 2 * D, 2 * K),
                         preferred_element_type=jnp.float32)
        part = jnp.sum(_softplus(scores)) * (1.0 / (B * K))

        @pl.when(i == 0)
        def _():
            out_ref[0, 0] = part

        @pl.when(i != 0)
        def _():
            out_ref[0, 0] += part

    return pl.pallas_call(
        body,
        grid=(nblk,),
        in_specs=[
            pl.BlockSpec((K, BLK, 2 * D), lambda i: (0, i, 0)),
            pl.BlockSpec((BLK, 2 * D), lambda i: (i, 0)),
        ],
        out_specs=pl.BlockSpec(memory_space=pltpu.SMEM),
        out_shape=jax.ShapeDtypeStruct((1, 1), jnp.float32),
    )(neg_pk, ce2_pk)


def _blockdiag2(W):
    n, m = W.shape
    z = jnp.zeros((n, m), jnp.float32)
    return jnp.concatenate([
        jnp.concatenate([W, z], axis=1),
        jnp.concatenate([z, W], axis=1),
    ], axis=0)


def kernel(center_word_indices, context_word_indices, negative_word_indices,
           center_emb, context_emb, W1, b1, W2, b2):
    B, K = negative_word_indices.shape
    V, D = center_emb.shape
    H = W1.shape[1]
    cidx = center_word_indices.astype(jnp.int32)
    xidx = context_word_indices.astype(jnp.int32)
    nidx = negative_word_indices.astype(jnp.int32).T.reshape(-1)

    def eo(a):
        return jnp.stack([a[0::2], a[1::2]])

    cemb_p = _tc_transpose_pad(center_emb.T, V, D)
    xemb_p = _tc_transpose_pad(context_emb.T, V, D)
    ce_pk, ct_pk, neg_pk = _sc_gather(eo(cidx), eo(xidx), eo(nidx),
                                      cemb_p, xemb_p, B, K, D)
    W1d = _blockdiag2(W1)
    W2d = _blockdiag2(W2)
    b1d = jnp.concatenate([b1, b1]).reshape(1, 2 * H)
    b2d = jnp.concatenate([b2, b2]).reshape(1, 2 * D)
    ce2_pk, pos_loss = _tc_mlp_pos(ce_pk, ct_pk, W1d, b1d, W2d, b2d, B, D, H)
    neg_loss = _tc_neg(neg_pk, ce2_pk, B, K, D)
    return pos_loss[0, 0] + neg_loss[0, 0]


# speedup vs baseline: 1.6336x; 1.0709x over previous
import functools

import jax
import jax.numpy as jnp
from jax import lax
from jax.experimental import pallas as pl
from jax.experimental.pallas import tpu as pltpu
from jax.experimental.pallas import tpu_sc as plsc
from jax.experimental import layout as jex_layout

NC, NS = 2, 16
NW = NC * NS
CH = 128
GROUP = 512


def _make_group(rows_e, rows_o, sem, D):
    def group(table_h, idx_ref, idx_off, dst, nh):
        cps = []
        for half, buf in ((0, rows_e), (1, rows_o)):
            for c in range(nh // CH):
                cps.append(pltpu.async_copy(
                    table_h.at[idx_ref.at[half,
                                          pl.ds(idx_off + c * CH, CH)]],
                    buf.at[pl.ds(c * CH, CH)], sem))
        for cp in cps:
            cp.wait()
        pltpu.sync_copy(rows_e.at[pl.ds(0, nh), pl.ds(0, D)],
                        dst.at[:, pl.ds(0, D)])
        pltpu.sync_copy(rows_o.at[pl.ds(0, nh), pl.ds(0, D)],
                        dst.at[:, pl.ds(D, D)])
    return group


def _sc_gather_ctx(xidx_eo, nidx_eo, xemb, B, K, D):
    BK = B * K
    Dp = 2 * D
    hpw = B // 2 // NW
    nhpw = BK // 2 // NW
    GH = GROUP // 2
    mesh = plsc.VectorSubcoreMesh(core_axis_name="c", subcore_axis_name="s")

    @functools.partial(
        pl.kernel,
        out_type=(
            jax.ShapeDtypeStruct((B // 2, 2 * D), jnp.float32),
            jax.ShapeDtypeStruct((K, B // 2, 2 * D), jnp.float32),
        ),
        mesh=mesh,
        compiler_params=pltpu.CompilerParams(use_tc_tiling_on_sc=False),
        scratch_types=[
            pltpu.VMEM((2, hpw), jnp.int32),
            pltpu.VMEM((2, nhpw), jnp.int32),
            pltpu.VMEM((GH, Dp), jnp.float32),
            pltpu.VMEM((GH, Dp), jnp.float32),
            pltpu.SemaphoreType.DMA,
        ],
    )
    def gather_kernel(xidx_h, nidx_h, xemb_h, ct_pk_o, ne_pk_o,
                      idx_x, idx_n, rows_e, rows_o, sem):
        wid = lax.axis_index("s") * NC + lax.axis_index("c")
        pltpu.sync_copy(xidx_h.at[:, pl.ds(wid * hpw, hpw)], idx_x)
        pltpu.sync_copy(nidx_h.at[:, pl.ds(wid * nhpw, nhpw)], idx_n)
        group = _make_group(rows_e, rows_o, sem, D)
        group(xemb_h, idx_x, 0,
              ct_pk_o.at[pl.ds(wid * hpw, hpw), :], hpw)
        for g in range(nhpw // GH):
            half_row = wid * nhpw + g * GH
            k = half_row // (B // 2)
            j0 = half_row % (B // 2)
            group(xemb_h, idx_n, g * GH,
                  ne_pk_o.at[k].at[pl.ds(j0, GH), :], GH)

    return gather_kernel(xidx_eo, nidx_eo, xemb)


def _sc_gather_ctr(cidx_eo, cemb, B, D):
    Dp = 2 * D
    hpw = B // 2 // NW
    GH = GROUP // 2
    mesh = plsc.VectorSubcoreMesh(core_axis_name="c", subcore_axis_name="s")

    @functools.partial(
        pl.kernel,
        out_type=jax.ShapeDtypeStruct((B // 2, 2 * D), jnp.float32),
        mesh=mesh,
        compiler_params=pltpu.CompilerParams(use_tc_tiling_on_sc=False),
        scratch_types=[
            pltpu.VMEM((2, hpw), jnp.int32),
            pltpu.VMEM((GH, Dp), jnp.float32),
            pltpu.VMEM((GH, Dp), jnp.float32),
            pltpu.SemaphoreType.DMA,
        ],
    )
    def gather_kernel(cidx_h, cemb_h, ce_pk_o, idx_c, rows_e, rows_o, sem):
        wid = lax.axis_index("s") * NC + lax.axis_index("c")
        pltpu.sync_copy(cidx_h.at[:, pl.ds(wid * hpw, hpw)], idx_c)
        group = _make_group(rows_e, rows_o, sem, D)
        group(cemb_h, idx_c, 0,
              ce_pk_o.at[pl.ds(wid * hpw, hpw), :], hpw)

    return gather_kernel(cidx_eo, cemb)


def _tc_transpose_pad(tableT, V, D):
    BLKV = 8192
    nblk = V // BLKV

    def body(t_ref, out_ref):
        x = t_ref[...]
        xt = jnp.swapaxes(x, 0, 1)
        out_ref[...] = jnp.concatenate([xt, jnp.zeros_like(xt)], axis=1)

    return pl.pallas_call(
        body,
        grid=(nblk,),
        in_specs=[pl.BlockSpec((D, BLKV), lambda i: (0, i))],
        out_specs=pl.BlockSpec((BLKV, 2 * D), lambda i: (i, 0)),
        out_shape=jax.ShapeDtypeStruct((V, 2 * D), jnp.float32),
    )(tableT)


def _softplus(x):
    return jnp.maximum(x, 0.0) + jnp.log1p(jnp.exp(-jnp.abs(x)))


def _half_mask(rows, cols):
    r = lax.broadcasted_iota(jnp.int32, (rows, cols), 0)
    c = lax.broadcasted_iota(jnp.int32, (rows, cols), 1)
    return jnp.where(r // 64 == c, 1.0, 0.0).astype(jnp.float32)


def _tc_mlp_pos(ce_pk, ct_pk, W1d, b1d, W2d, b2d, B, D, H):
    BLK = 512
    npk = B // 2
    nblk = npk // BLK

    def body(ce_ref, ct_ref, w1_ref, b1_ref, w2_ref, b2_ref,
             ce2_ref, pos_ref):
        i = pl.program_id(0)
        h = jnp.dot(ce_ref[...], w1_ref[...],
                    preferred_element_type=jnp.float32) + b1_ref[...]
        h = jnp.maximum(h, 0.0)
        ce2 = jnp.dot(h, w2_ref[...],
                      preferred_element_type=jnp.float32) + b2_ref[...]
        ce2_ref[...] = ce2
        prod = ce2 * ct_ref[...]
        pos = jnp.dot(prod, _half_mask(2 * D, 2),
                      preferred_element_type=jnp.float32)
        part = jnp.sum(_softplus(-pos)) * (1.0 / B)

        @pl.when(i == 0)
        def _():
            pos_ref[0, 0] = part

        @pl.when(i != 0)
        def _():
            pos_ref[0, 0] += part

    return pl.pallas_call(
        body,
        grid=(nblk,),
        in_specs=[
            pl.BlockSpec((BLK, 2 * D), lambda i: (i, 0)),
            pl.BlockSpec((BLK, 2 * D), lambda i: (i, 0)),
            pl.BlockSpec((2 * D, 2 * H), lambda i: (0, 0)),
            pl.BlockSpec((1, 2 * H), lambda i: (0, 0)),
            pl.BlockSpec((2 * H, 2 * D), lambda i: (0, 0)),
            pl.BlockSpec((1, 2 * D), lambda i: (0, 0)),
        ],
        out_specs=[
            pl.BlockSpec((BLK, 2 * D), lambda i: (i, 0)),
            pl.BlockSpec(memory_space=pltpu.SMEM),
        ],
        out_shape=[
            jax.ShapeDtypeStruct((npk, 2 * D), jnp.float32),
            jax.ShapeDtypeStruct((1, 1), jnp.float32),
        ],
    )(ce_pk, ct_pk, W1d, b1d, W2d, b2d)


def _tc_neg(neg_pk, ce2_pk, B, K, D):
    BLK = 512
    npk = B // 2
    nblk = npk // BLK

    def body(ne_ref, ce2_ref, out_ref):
        i = pl.program_id(0)
        ce2 = ce2_ref[...]
        prods = [ne_ref[k] * ce2 for k in range(K)]
        pall = jnp.concatenate(prods, axis=1)
        scores = jnp.dot(pall, _half_mask(K * 2 * D, 2 * K),
                         preferred_element_type=jnp.float32)
        part = jnp.sum(_softplus(scores)) * (1.0 / (B * K))

        @pl.when(i == 0)
        def _():
            out_ref[0, 0] = part

        @pl.when(i != 0)
        def _():
            out_ref[0, 0] += part

    return pl.pallas_call(
        body,
        grid=(nblk,),
        in_specs=[
            pl.BlockSpec((K, BLK, 2 * D), lambda i: (0, i, 0)),
            pl.BlockSpec((BLK, 2 * D), lambda i: (i, 0)),
        ],
        out_specs=pl.BlockSpec(memory_space=pltpu.SMEM),
        out_shape=jax.ShapeDtypeStruct((1, 1), jnp.float32),
    )(neg_pk, ce2_pk)


def _blockdiag2(W):
    n, m = W.shape
    z = jnp.zeros((n, m), jnp.float32)
    return jnp.concatenate([
        jnp.concatenate([W, z], axis=1),
        jnp.concatenate([z, W], axis=1),
    ], axis=0)


def kernel(center_word_indices, context_word_indices, negative_word_indices,
           center_emb, context_emb, W1, b1, W2, b2):
    B, K = negative_word_indices.shape
    V, D = center_emb.shape
    H = W1.shape[1]
    cidx = center_word_indices.astype(jnp.int32)
    xidx = context_word_indices.astype(jnp.int32)
    nidx = negative_word_indices.astype(jnp.int32).T.reshape(-1)

    def eo(a):
        return jnp.stack([a[0::2], a[1::2]])

    xemb_p = _tc_transpose_pad(context_emb.T, V, D)
    ct_pk, neg_pk = _sc_gather_ctx(eo(xidx), eo(nidx), xemb_p, B, K, D)
    cemb_p = _tc_transpose_pad(center_emb.T, V, D)
    ce_pk = _sc_gather_ctr(eo(cidx), cemb_p, B, D)
    W1d = _blockdiag2(W1)
    W2d = _blockdiag2(W2)
    b1d = jnp.concatenate([b1, b1]).reshape(1, 2 * H)
    b2d = jnp.concatenate([b2, b2]).reshape(1, 2 * D)
    ce2_pk, pos_loss = _tc_mlp_pos(ce_pk, ct_pk, W1d, b1d, W2d, b2d, B, D, H)
    neg_loss = _tc_neg(neg_pk, ce2_pk, B, K, D)
    return pos_loss[0, 0] + neg_loss[0, 0]


# speedup vs baseline: 1.7610x; 1.0780x over previous
import functools

import jax
import jax.numpy as jnp
from jax import lax
from jax.experimental import pallas as pl
from jax.experimental.pallas import tpu as pltpu
from jax.experimental.pallas import tpu_sc as plsc
from jax.experimental import layout as jex_layout

NC, NS = 2, 16
NW = NC * NS
CH = 128
GROUP = 512


def _make_group(rows_e, rows_o, sem, D):
    def group(table_h, idx_ref, idx_off, dst, nh):
        cps = []
        for half, buf in ((0, rows_e), (1, rows_o)):
            for c in range(nh // CH):
                cps.append(pltpu.async_copy(
                    table_h.at[idx_ref.at[half,
                                          pl.ds(idx_off + c * CH, CH)]],
                    buf.at[pl.ds(c * CH, CH)], sem))
        for cp in cps:
            cp.wait()
        pltpu.sync_copy(rows_e.at[pl.ds(0, nh), pl.ds(0, D)],
                        dst.at[:, pl.ds(0, D)])
        pltpu.sync_copy(rows_o.at[pl.ds(0, nh), pl.ds(0, D)],
                        dst.at[:, pl.ds(D, D)])
    return group


def _sc_gather_ctx(xidx_eo, nidx_eo, xemb, B, K, D):
    BK = B * K
    Dp = 2 * D
    hpw = B // 2 // NW
    nhpw = BK // 2 // NW
    GH = GROUP // 2
    mesh = plsc.VectorSubcoreMesh(core_axis_name="c", subcore_axis_name="s")

    @functools.partial(
        pl.kernel,
        out_type=(
            jax.ShapeDtypeStruct((B // 2, 2 * D), jnp.float32),
            jax.ShapeDtypeStruct((K, B // 2, 2 * D), jnp.float32),
        ),
        mesh=mesh,
        compiler_params=pltpu.CompilerParams(use_tc_tiling_on_sc=False),
        scratch_types=[
            pltpu.VMEM((2, hpw), jnp.int32),
            pltpu.VMEM((2, nhpw), jnp.int32),
            pltpu.VMEM((GH, Dp), jnp.float32),
            pltpu.VMEM((GH, Dp), jnp.float32),
            pltpu.SemaphoreType.DMA,
        ],
    )
    def gather_kernel(xidx_h, nidx_h, xemb_h, ct_pk_o, ne_pk_o,
                      idx_x, idx_n, rows_e, rows_o, sem):
        wid = lax.axis_index("s") * NC + lax.axis_index("c")
        pltpu.sync_copy(xidx_h.at[:, pl.ds(wid * hpw, hpw)], idx_x)
        pltpu.sync_copy(nidx_h.at[:, pl.ds(wid * nhpw, nhpw)], idx_n)
        group = _make_group(rows_e, rows_o, sem, D)
        group(xemb_h, idx_x, 0,
              ct_pk_o.at[pl.ds(wid * hpw, hpw), :], hpw)
        for g in range(nhpw // GH):
            half_row = wid * nhpw + g * GH
            k = half_row // (B // 2)
            j0 = half_row % (B // 2)
            group(xemb_h, idx_n, g * GH,
                  ne_pk_o.at[k].at[pl.ds(j0, GH), :], GH)

    return gather_kernel(xidx_eo, nidx_eo, xemb)


def _sc_gather_ctr(cidx_eo, cemb, B, D):
    Dp = 2 * D
    hpw = B // 2 // NW
    GH = GROUP // 2
    mesh = plsc.VectorSubcoreMesh(core_axis_name="c", subcore_axis_name="s")

    @functools.partial(
        pl.kernel,
        out_type=jax.ShapeDtypeStruct((B // 2, 2 * D), jnp.float32),
        mesh=mesh,
        compiler_params=pltpu.CompilerParams(use_tc_tiling_on_sc=False),
        scratch_types=[
            pltpu.VMEM((2, hpw), jnp.int32),
            pltpu.VMEM((GH, Dp), jnp.float32),
            pltpu.VMEM((GH, Dp), jnp.float32),
            pltpu.SemaphoreType.DMA,
        ],
    )
    def gather_kernel(cidx_h, cemb_h, ce_pk_o, idx_c, rows_e, rows_o, sem):
        wid = lax.axis_index("s") * NC + lax.axis_index("c")
        pltpu.sync_copy(cidx_h.at[:, pl.ds(wid * hpw, hpw)], idx_c)
        group = _make_group(rows_e, rows_o, sem, D)
        group(cemb_h, idx_c, 0,
              ce_pk_o.at[pl.ds(wid * hpw, hpw), :], hpw)

    return gather_kernel(cidx_eo, cemb)


def _tc_transpose_pad(tableT, V, D):
    BLKV = 8192
    nblk = pl.cdiv(V, BLKV)

    def body(t_ref, out_ref):
        x = t_ref[...]
        xt = jnp.swapaxes(x, 0, 1)
        out_ref[...] = jnp.concatenate([xt, jnp.zeros_like(xt)], axis=1)

    return pl.pallas_call(
        body,
        grid=(nblk,),
        in_specs=[pl.BlockSpec((D, BLKV), lambda i: (0, i))],
        out_specs=pl.BlockSpec((BLKV, 2 * D), lambda i: (i, 0)),
        out_shape=jax.ShapeDtypeStruct((V, 2 * D), jnp.float32),
    )(tableT)


def _softplus(x):
    return jnp.maximum(x, 0.0) + jnp.log1p(jnp.exp(-jnp.abs(x)))


def _half_mask(rows, cols):
    r = lax.broadcasted_iota(jnp.int32, (rows, cols), 0)
    c = lax.broadcasted_iota(jnp.int32, (rows, cols), 1)
    return jnp.where(r // 64 == c, 1.0, 0.0).astype(jnp.float32)


def _tc_mlp_pos(ce_pk, ct_pk, W1d, b1d, W2d, b2d, B, D, H):
    BLK = 512
    npk = B // 2
    nblk = npk // BLK

    def body(ce_ref, ct_ref, w1_ref, b1_ref, w2_ref, b2_ref,
             ce2_ref, pos_ref):
        i = pl.program_id(0)
        h = jnp.dot(ce_ref[...], w1_ref[...],
                    preferred_element_type=jnp.float32) + b1_ref[...]
        h = jnp.maximum(h, 0.0)
        ce2 = jnp.dot(h, w2_ref[...],
                      preferred_element_type=jnp.float32) + b2_ref[...]
        ce2_ref[...] = ce2
        prod = ce2 * ct_ref[...]
        pos = jnp.dot(prod, _half_mask(2 * D, 2),
                      preferred_element_type=jnp.float32)
        part = jnp.sum(_softplus(-pos)) * (1.0 / B)

        @pl.when(i == 0)
        def _():
            pos_ref[0, 0] = part

        @pl.when(i != 0)
        def _():
            pos_ref[0, 0] += part

    return pl.pallas_call(
        body,
        grid=(nblk,),
        in_specs=[
            pl.BlockSpec((BLK, 2 * D), lambda i: (i, 0)),
            pl.BlockSpec((BLK, 2 * D), lambda i: (i, 0)),
            pl.BlockSpec((2 * D, 2 * H), lambda i: (0, 0)),
            pl.BlockSpec((1, 2 * H), lambda i: (0, 0)),
            pl.BlockSpec((2 * H, 2 * D), lambda i: (0, 0)),
            pl.BlockSpec((1, 2 * D), lambda i: (0, 0)),
        ],
        out_specs=[
            pl.BlockSpec((BLK, 2 * D), lambda i: (i, 0)),
            pl.BlockSpec(memory_space=pltpu.SMEM),
        ],
        out_shape=[
            jax.ShapeDtypeStruct((npk, 2 * D), jnp.float32),
            jax.ShapeDtypeStruct((1, 1), jnp.float32),
        ],
    )(ce_pk, ct_pk, W1d, b1d, W2d, b2d)


def _tc_neg(neg_pk, ce2_pk, B, K, D):
    BLK = 512
    npk = B // 2
    nblk = npk // BLK

    def body(ne_ref, ce2_ref, out_ref):
        i = pl.program_id(0)
        ce2 = ce2_ref[...]
        prods = [ne_ref[k] * ce2 for k in range(K)]
        pall = jnp.concatenate(prods, axis=1)
        scores = jnp.dot(pall, _half_mask(K * 2 * D, 2 * K),
                         preferred_element_type=jnp.float32)
        part = jnp.sum(_softplus(scores)) * (1.0 / (B * K))

        @pl.when(i == 0)
        def _():
            out_ref[0, 0] = part

        @pl.when(i != 0)
        def _():
            out_ref[0, 0] += part

    return pl.pallas_call(
        body,
        grid=(nblk,),
        in_specs=[
            pl.BlockSpec((K, BLK, 2 * D), lambda i: (0, i, 0)),
            pl.BlockSpec((BLK, 2 * D), lambda i: (i, 0)),
        ],
        out_specs=pl.BlockSpec(memory_space=pltpu.SMEM),
        out_shape=jax.ShapeDtypeStruct((1, 1), jnp.float32),
    )(neg_pk, ce2_pk)


def _blockdiag2(W):
    n, m = W.shape
    z = jnp.zeros((n, m), jnp.float32)
    return jnp.concatenate([
        jnp.concatenate([W, z], axis=1),
        jnp.concatenate([z, W], axis=1),
    ], axis=0)


def kernel(center_word_indices, context_word_indices, negative_word_indices,
           center_emb, context_emb, W1, b1, W2, b2):
    B, K = negative_word_indices.shape
    V, D = center_emb.shape
    H = W1.shape[1]
    cidx = center_word_indices.astype(jnp.int32)
    xidx = context_word_indices.astype(jnp.int32)
    nidx2 = negative_word_indices.astype(jnp.int32).T

    def halves(a):
        return jnp.stack([a[: B // 2], a[B // 2:]])

    nidx_eo = jnp.stack([nidx2[:, : B // 2].reshape(-1),
                         nidx2[:, B // 2:].reshape(-1)])

    xemb_p = _tc_transpose_pad(context_emb.T, V, D)
    ct_pk, neg_pk = _sc_gather_ctx(halves(xidx), nidx_eo, xemb_p, B, K, D)
    cemb_p = _tc_transpose_pad(center_emb.T, V, D)
    ce_pk = _sc_gather_ctr(halves(cidx), cemb_p, B, D)
    W1d = _blockdiag2(W1)
    W2d = _blockdiag2(W2)
    b1d = jnp.concatenate([b1, b1]).reshape(1, 2 * H)
    b2d = jnp.concatenate([b2, b2]).reshape(1, 2 * D)
    ce2_pk, pos_loss = _tc_mlp_pos(ce_pk, ct_pk, W1d, b1d, W2d, b2d, B, D, H)
    neg_loss = _tc_neg(neg_pk, ce2_pk, B, K, D)
    return pos_loss[0, 0] + neg_loss[0, 0]


# speedup vs baseline: 1.8637x; 1.0584x over previous
import functools

import jax
import jax.numpy as jnp
from jax import lax
from jax.experimental import pallas as pl
from jax.experimental.pallas import tpu as pltpu
from jax.experimental.pallas import tpu_sc as plsc
from jax.experimental import layout as jex_layout

NC, NS = 2, 16
NW = NC * NS
CH = 128
GROUP = 512


def _make_group(rows_e, rows_o, sem, D):
    def group(table_h, idx_ref, idx_off, dst, nh):
        cps = []
        for half, buf in ((0, rows_e), (1, rows_o)):
            for c in range(nh // CH):
                cps.append(pltpu.async_copy(
                    table_h.at[idx_ref.at[half,
                                          pl.ds(idx_off + c * CH, CH)]],
                    buf.at[pl.ds(c * CH, CH)], sem))
        for cp in cps:
            cp.wait()
        pltpu.sync_copy(rows_e.at[pl.ds(0, nh), pl.ds(0, D)],
                        dst.at[:, pl.ds(0, D)])
        pltpu.sync_copy(rows_o.at[pl.ds(0, nh), pl.ds(0, D)],
                        dst.at[:, pl.ds(D, D)])
    return group


def _sc_gather_ctx(xidx_eo, nidx_eo, xemb, B, K, D):
    BK = B * K
    Dp = 2 * D
    hpw = B // 2 // NW
    nhpw = BK // 2 // NW
    GH = GROUP // 2
    mesh = plsc.VectorSubcoreMesh(core_axis_name="c", subcore_axis_name="s")

    @functools.partial(
        pl.kernel,
        out_type=(
            jax.ShapeDtypeStruct((B // 2, 2 * D), jnp.float32),
            jax.ShapeDtypeStruct((K, B // 2, 2 * D), jnp.float32),
        ),
        mesh=mesh,
        compiler_params=pltpu.CompilerParams(use_tc_tiling_on_sc=False),
        scratch_types=[
            pltpu.VMEM((2, hpw), jnp.int32),
            pltpu.VMEM((2, nhpw), jnp.int32),
            pltpu.VMEM((GH, Dp), jnp.float32),
            pltpu.VMEM((GH, Dp), jnp.float32),
            pltpu.SemaphoreType.DMA,
        ],
    )
    def gather_kernel(xidx_h, nidx_h, xemb_h, ct_pk_o, ne_pk_o,
                      idx_x, idx_n, rows_e, rows_o, sem):
        wid = lax.axis_index("s") * NC + lax.axis_index("c")
        pltpu.sync_copy(xidx_h.at[:, pl.ds(wid * hpw, hpw)], idx_x)
        pltpu.sync_copy(nidx_h.at[:, pl.ds(wid * nhpw, nhpw)], idx_n)
        group = _make_group(rows_e, rows_o, sem, D)
        group(xemb_h, idx_x, 0,
              ct_pk_o.at[pl.ds(wid * hpw, hpw), :], hpw)
        for g in range(nhpw // GH):
            half_row = wid * nhpw + g * GH
            k = half_row // (B // 2)
            j0 = half_row % (B // 2)
            group(xemb_h, idx_n, g * GH,
                  ne_pk_o.at[k].at[pl.ds(j0, GH), :], GH)

    return gather_kernel(xidx_eo, nidx_eo, xemb)


def _sc_gather_ctr(cidx_eo, cemb, B, D):
    Dp = 2 * D
    hpw = B // 2 // NW
    GH = GROUP // 2
    mesh = plsc.VectorSubcoreMesh(core_axis_name="c", subcore_axis_name="s")

    @functools.partial(
        pl.kernel,
        out_type=jax.ShapeDtypeStruct((B // 2, 2 * D), jnp.float32),
        mesh=mesh,
        compiler_params=pltpu.CompilerParams(use_tc_tiling_on_sc=False),
        scratch_types=[
            pltpu.VMEM((2, hpw), jnp.int32),
            pltpu.VMEM((GH, Dp), jnp.float32),
            pltpu.VMEM((GH, Dp), jnp.float32),
            pltpu.SemaphoreType.DMA,
        ],
    )
    def gather_kernel(cidx_h, cemb_h, ce_pk_o, idx_c, rows_e, rows_o, sem):
        wid = lax.axis_index("s") * NC + lax.axis_index("c")
        pltpu.sync_copy(cidx_h.at[:, pl.ds(wid * hpw, hpw)], idx_c)
        group = _make_group(rows_e, rows_o, sem, D)
        group(cemb_h, idx_c, 0,
              ce_pk_o.at[pl.ds(wid * hpw, hpw), :], hpw)

    return gather_kernel(cidx_eo, cemb)


def _tc_transpose_pad(tableT, V, D):
    BLKV = 16384
    nblk = pl.cdiv(V, BLKV)

    def body(t_ref, out_ref):
        x = t_ref[...]
        xt = jnp.swapaxes(x, 0, 1)
        out_ref[...] = jnp.concatenate([xt, jnp.zeros_like(xt)], axis=1)

    return pl.pallas_call(
        body,
        grid=(nblk,),
        in_specs=[pl.BlockSpec((D, BLKV), lambda i: (0, i))],
        out_specs=pl.BlockSpec((BLKV, 2 * D), lambda i: (i, 0)),
        out_shape=jax.ShapeDtypeStruct((V, 2 * D), jnp.float32),
    )(tableT)


def _softplus(x):
    return jnp.maximum(x, 0.0) + jnp.log1p(jnp.exp(-jnp.abs(x)))


def _half_mask(rows, cols):
    r = lax.broadcasted_iota(jnp.int32, (rows, cols), 0)
    c = lax.broadcasted_iota(jnp.int32, (rows, cols), 1)
    return jnp.where(r // 64 == c, 1.0, 0.0).astype(jnp.float32)


def _tc_mlp_pos(ce_pk, ct_pk, W1d, b1d, W2d, b2d, B, D, H):
    BLK = 512
    npk = B // 2
    nblk = npk // BLK

    def body(ce_ref, ct_ref, w1_ref, b1_ref, w2_ref, b2_ref,
             ce2_ref, pos_ref):
        i = pl.program_id(0)
        h = jnp.dot(ce_ref[...], w1_ref[...],
                    preferred_element_type=jnp.float32) + b1_ref[...]
        h = jnp.maximum(h, 0.0)
        ce2 = jnp.dot(h, w2_ref[...],
                      preferred_element_type=jnp.float32) + b2_ref[...]
        ce2_ref[...] = ce2
        prod = ce2 * ct_ref[...]
        pos = jnp.dot(prod, _half_mask(2 * D, 2),
                      preferred_element_type=jnp.float32)
        part = jnp.sum(_softplus(-pos)) * (1.0 / B)

        @pl.when(i == 0)
        def _():
            pos_ref[0, 0] = part

        @pl.when(i != 0)
        def _():
            pos_ref[0, 0] += part

    return pl.pallas_call(
        body,
        grid=(nblk,),
        in_specs=[
            pl.BlockSpec((BLK, 2 * D), lambda i: (i, 0)),
            pl.BlockSpec((BLK, 2 * D), lambda i: (i, 0)),
            pl.BlockSpec((2 * D, 2 * H), lambda i: (0, 0)),
            pl.BlockSpec((1, 2 * H), lambda i: (0, 0)),
            pl.BlockSpec((2 * H, 2 * D), lambda i: (0, 0)),
            pl.BlockSpec((1, 2 * D), lambda i: (0, 0)),
        ],
        out_specs=[
            pl.BlockSpec((BLK, 2 * D), lambda i: (i, 0)),
            pl.BlockSpec(memory_space=pltpu.SMEM),
        ],
        out_shape=[
            jax.ShapeDtypeStruct((npk, 2 * D), jnp.float32),
            jax.ShapeDtypeStruct((1, 1), jnp.float32),
        ],
    )(ce_pk, ct_pk, W1d, b1d, W2d, b2d)


def _tc_neg(neg_pk, ce2_pk, B, K, D):
    BLK = 512
    npk = B // 2
    nblk = npk // BLK

    def body(ne_ref, ce2_ref, out_ref):
        i = pl.program_id(0)
        ce2 = ce2_ref[...]
        prods = [ne_ref[k] * ce2 for k in range(K)]
        pall = jnp.concatenate(prods, axis=1)
        scores = jnp.dot(pall, _half_mask(K * 2 * D, 2 * K),
                         preferred_element_type=jnp.float32)
        part = jnp.sum(_softplus(scores)) * (1.0 / (B * K))

        @pl.when(i == 0)
        def _():
            out_ref[0, 0] = part

        @pl.when(i != 0)
        def _():
            out_ref[0, 0] += part

    return pl.pallas_call(
        body,
        grid=(nblk,),
        in_specs=[
            pl.BlockSpec((K, BLK, 2 * D), lambda i: (0, i, 0)),
            pl.BlockSpec((BLK, 2 * D), lambda i: (i, 0)),
        ],
        out_specs=pl.BlockSpec(memory_space=pltpu.SMEM),
        out_shape=jax.ShapeDtypeStruct((1, 1), jnp.float32),
    )(neg_pk, ce2_pk)


def _blockdiag2(W):
    n, m = W.shape
    z = jnp.zeros((n, m), jnp.float32)
    return jnp.concatenate([
        jnp.concatenate([W, z], axis=1),
        jnp.concatenate([z, W], axis=1),
    ], axis=0)


def kernel(center_word_indices, context_word_indices, negative_word_indices,
           center_emb, context_emb, W1, b1, W2, b2):
    B, K = negative_word_indices.shape
    V, D = center_emb.shape
    H = W1.shape[1]
    cidx = center_word_indices.astype(jnp.int32)
    xidx = context_word_indices.astype(jnp.int32)
    nidx2 = negative_word_indices.astype(jnp.int32).T

    def halves(a):
        return jnp.stack([a[: B // 2], a[B // 2:]])

    nidx_eo = jnp.stack([nidx2[:, : B // 2].reshape(-1),
                         nidx2[:, B // 2:].reshape(-1)])

    xemb_p = _tc_transpose_pad(context_emb.T, V, D)
    ct_pk, neg_pk = _sc_gather_ctx(halves(xidx), nidx_eo, xemb_p, B, K, D)
    cemb_p = _tc_transpose_pad(center_emb.T, V, D)
    ce_pk = _sc_gather_ctr(halves(cidx), cemb_p, B, D)
    W1d = _blockdiag2(W1)
    W2d = _blockdiag2(W2)
    b1d = jnp.concatenate([b1, b1]).reshape(1, 2 * H)
    b2d = jnp.concatenate([b2, b2]).reshape(1, 2 * D)
    ce2_pk, pos_loss = _tc_mlp_pos(ce_pk, ct_pk, W1d, b1d, W2d, b2d, B, D, H)
    neg_loss = _tc_neg(neg_pk, ce2_pk, B, K, D)
    return pos_loss[0, 0] + neg_loss[0, 0]


# speedup vs baseline: 1.8898x; 1.0140x over previous
import functools

import jax
import jax.numpy as jnp
from jax import lax
from jax.experimental import pallas as pl
from jax.experimental.pallas import tpu as pltpu
from jax.experimental.pallas import tpu_sc as plsc
from jax.experimental import layout as jex_layout

NC, NS = 2, 16
NW = NC * NS
CH = 128
GROUP = 512


def _make_group(rows_e, rows_o, sem, D):
    def group(table_h, idx_ref, idx_off, dst, nh):
        cps = []
        for half, buf in ((0, rows_e), (1, rows_o)):
            for c in range(nh // CH):
                cps.append(pltpu.async_copy(
                    table_h.at[idx_ref.at[half,
                                          pl.ds(idx_off + c * CH, CH)]],
                    buf.at[pl.ds(c * CH, CH)], sem))
        for cp in cps:
            cp.wait()
        pltpu.sync_copy(rows_e.at[pl.ds(0, nh), pl.ds(0, D)],
                        dst.at[:, pl.ds(0, D)])
        pltpu.sync_copy(rows_o.at[pl.ds(0, nh), pl.ds(0, D)],
                        dst.at[:, pl.ds(D, D)])
    return group


def _sc_gather_ctx(xidx_eo, nidx_eo, xemb, B, K, D):
    BK = B * K
    Dp = 2 * D
    hpw = B // 2 // NW
    nhpw = BK // 2 // NW
    GH = GROUP // 2
    mesh = plsc.VectorSubcoreMesh(core_axis_name="c", subcore_axis_name="s")

    @functools.partial(
        pl.kernel,
        out_type=(
            jax.ShapeDtypeStruct((B // 2, 2 * D), jnp.float32),
            jax.ShapeDtypeStruct((K, B // 2, 2 * D), jnp.float32),
        ),
        mesh=mesh,
        compiler_params=pltpu.CompilerParams(use_tc_tiling_on_sc=False),
        scratch_types=[
            pltpu.VMEM((2, hpw), jnp.int32),
            pltpu.VMEM((2, nhpw), jnp.int32),
            pltpu.VMEM((GH, Dp), jnp.float32),
            pltpu.VMEM((GH, Dp), jnp.float32),
            pltpu.SemaphoreType.DMA,
        ],
    )
    def gather_kernel(xidx_h, nidx_h, xemb_h, ct_pk_o, ne_pk_o,
                      idx_x, idx_n, rows_e, rows_o, sem):
        wid = lax.axis_index("s") * NC + lax.axis_index("c")
        pltpu.sync_copy(xidx_h.at[:, pl.ds(wid * hpw, hpw)], idx_x)
        pltpu.sync_copy(nidx_h.at[:, pl.ds(wid * nhpw, nhpw)], idx_n)
        group = _make_group(rows_e, rows_o, sem, D)
        group(xemb_h, idx_x, 0,
              ct_pk_o.at[pl.ds(wid * hpw, hpw), :], hpw)
        for g in range(nhpw // GH):
            half_row = wid * nhpw + g * GH
            k = half_row // (B // 2)
            j0 = half_row % (B // 2)
            group(xemb_h, idx_n, g * GH,
                  ne_pk_o.at[k].at[pl.ds(j0, GH), :], GH)

    return gather_kernel(xidx_eo, nidx_eo, xemb)


def _sc_gather_ctr(cidx_eo, cemb, B, D):
    Dp = 2 * D
    hpw = B // 2 // NW
    GH = GROUP // 2
    mesh = plsc.VectorSubcoreMesh(core_axis_name="c", subcore_axis_name="s")

    @functools.partial(
        pl.kernel,
        out_type=jax.ShapeDtypeStruct((B // 2, 2 * D), jnp.float32),
        mesh=mesh,
        compiler_params=pltpu.CompilerParams(use_tc_tiling_on_sc=False),
        scratch_types=[
            pltpu.VMEM((2, hpw), jnp.int32),
            pltpu.VMEM((GH, Dp), jnp.float32),
            pltpu.VMEM((GH, Dp), jnp.float32),
            pltpu.SemaphoreType.DMA,
        ],
    )
    def gather_kernel(cidx_h, cemb_h, ce_pk_o, idx_c, rows_e, rows_o, sem):
        wid = lax.axis_index("s") * NC + lax.axis_index("c")
        pltpu.sync_copy(cidx_h.at[:, pl.ds(wid * hpw, hpw)], idx_c)
        group = _make_group(rows_e, rows_o, sem, D)
        group(cemb_h, idx_c, 0,
              ce_pk_o.at[pl.ds(wid * hpw, hpw), :], hpw)

    return gather_kernel(cidx_eo, cemb)


def _tc_transpose_pad(tableT, V, D):
    BLKV = 32768
    nblk = pl.cdiv(V, BLKV)

    def body(t_ref, out_ref):
        x = t_ref[...]
        xt = jnp.swapaxes(x, 0, 1)
        out_ref[...] = jnp.concatenate([xt, jnp.zeros_like(xt)], axis=1)

    return pl.pallas_call(
        body,
        grid=(nblk,),
        in_specs=[pl.BlockSpec((D, BLKV), lambda i: (0, i))],
        out_specs=pl.BlockSpec((BLKV, 2 * D), lambda i: (i, 0)),
        out_shape=jax.ShapeDtypeStruct((V, 2 * D), jnp.float32),
    )(tableT)


def _softplus(x):
    return jnp.maximum(x, 0.0) + jnp.log1p(jnp.exp(-jnp.abs(x)))


def _half_mask(rows, cols):
    r = lax.broadcasted_iota(jnp.int32, (rows, cols), 0)
    c = lax.broadcasted_iota(jnp.int32, (rows, cols), 1)
    return jnp.where(r // 64 == c, 1.0, 0.0).astype(jnp.float32)


def _tc_mlp_pos(ce_pk, ct_pk, W1d, b1d, W2d, b2d, B, D, H):
    BLK = 512
    npk = B // 2
    nblk = npk // BLK

    def body(ce_ref, ct_ref, w1_ref, b1_ref, w2_ref, b2_ref,
             ce2_ref, pos_ref):
        i = pl.program_id(0)
        h = jnp.dot(ce_ref[...], w1_ref[...],
                    preferred_element_type=jnp.float32) + b1_ref[...]
        h = jnp.maximum(h, 0.0)
        ce2 = jnp.dot(h, w2_ref[...],
                      preferred_element_type=jnp.float32) + b2_ref[...]
        ce2_ref[...] = ce2
        prod = ce2 * ct_ref[...]
        pos = jnp.dot(prod, _half_mask(2 * D, 2),
                      preferred_element_type=jnp.float32)
        part = jnp.sum(_softplus(-pos)) * (1.0 / B)

        @pl.when(i == 0)
        def _():
            pos_ref[0, 0] = part

        @pl.when(i != 0)
        def _():
            pos_ref[0, 0] += part

    return pl.pallas_call(
        body,
        grid=(nblk,),
        in_specs=[
            pl.BlockSpec((BLK, 2 * D), lambda i: (i, 0)),
            pl.BlockSpec((BLK, 2 * D), lambda i: (i, 0)),
            pl.BlockSpec((2 * D, 2 * H), lambda i: (0, 0)),
            pl.BlockSpec((1, 2 * H), lambda i: (0, 0)),
            pl.BlockSpec((2 * H, 2 * D), lambda i: (0, 0)),
            pl.BlockSpec((1, 2 * D), lambda i: (0, 0)),
        ],
        out_specs=[
            pl.BlockSpec((BLK, 2 * D), lambda i: (i, 0)),
            pl.BlockSpec(memory_space=pltpu.SMEM),
        ],
        out_shape=[
            jax.ShapeDtypeStruct((npk, 2 * D), jnp.float32),
            jax.ShapeDtypeStruct((1, 1), jnp.float32),
        ],
    )(ce_pk, ct_pk, W1d, b1d, W2d, b2d)


def _tc_neg(neg_pk, ce2_pk, B, K, D):
    BLK = 512
    npk = B // 2
    nblk = npk // BLK

    def body(ne_ref, ce2_ref, out_ref):
        i = pl.program_id(0)
        ce2 = ce2_ref[...]
        prods = [ne_ref[k] * ce2 for k in range(K)]
        pall = jnp.concatenate(prods, axis=1)
        scores = jnp.dot(pall, _half_mask(K * 2 * D, 2 * K),
                         preferred_element_type=jnp.float32)
        part = jnp.sum(_softplus(scores)) * (1.0 / (B * K))

        @pl.when(i == 0)
        def _():
            out_ref[0, 0] = part

        @pl.when(i != 0)
        def _():
            out_ref[0, 0] += part

    return pl.pallas_call(
        body,
        grid=(nblk,),
        in_specs=[
            pl.BlockSpec((K, BLK, 2 * D), lambda i: (0, i, 0)),
            pl.BlockSpec((BLK, 2 * D), lambda i: (i, 0)),
        ],
        out_specs=pl.BlockSpec(memory_space=pltpu.SMEM),
        out_shape=jax.ShapeDtypeStruct((1, 1), jnp.float32),
    )(neg_pk, ce2_pk)


def _blockdiag2(W):
    n, m = W.shape
    z = jnp.zeros((n, m), jnp.float32)
    return jnp.concatenate([
        jnp.concatenate([W, z], axis=1),
        jnp.concatenate([z, W], axis=1),
    ], axis=0)


def kernel(center_word_indices, context_word_indices, negative_word_indices,
           center_emb, context_emb, W1, b1, W2, b2):
    B, K = negative_word_indices.shape
    V, D = center_emb.shape
    H = W1.shape[1]
    cidx = center_word_indices.astype(jnp.int32)
    xidx = context_word_indices.astype(jnp.int32)
    nidx2 = negative_word_indices.astype(jnp.int32).T

    def halves(a):
        return jnp.stack([a[: B // 2], a[B // 2:]])

    nidx_eo = jnp.stack([nidx2[:, : B // 2].reshape(-1),
                         nidx2[:, B // 2:].reshape(-1)])

    xemb_p = _tc_transpose_pad(context_emb.T, V, D)
    ct_pk, neg_pk = _sc_gather_ctx(halves(xidx), nidx_eo, xemb_p, B, K, D)
    cemb_p = _tc_transpose_pad(center_emb.T, V, D)
    ce_pk = _sc_gather_ctr(halves(cidx), cemb_p, B, D)
    W1d = _blockdiag2(W1)
    W2d = _blockdiag2(W2)
    b1d = jnp.concatenate([b1, b1]).reshape(1, 2 * H)
    b2d = jnp.concatenate([b2, b2]).reshape(1, 2 * D)
    ce2_pk, pos_loss = _tc_mlp_pos(ce_pk, ct_pk, W1d, b1d, W2d, b2d, B, D, H)
    neg_loss = _tc_neg(neg_pk, ce2_pk, B, K, D)
    return pos_loss[0, 0] + neg_loss[0, 0]


# speedup vs baseline: 1.8899x; 1.0001x over previous
import functools

import jax
import jax.numpy as jnp
from jax import lax
from jax.experimental import pallas as pl
from jax.experimental.pallas import tpu as pltpu
from jax.experimental.pallas import tpu_sc as plsc

NC, NS = 2, 16
NW = NC * NS
CH = 128
GROUP = 512


def _make_group(rows_e, rows_o, sem, D):
    def group(table_h, idx_ref, idx_off, dst, nh):
        cps = []
        for half, buf in ((0, rows_e), (1, rows_o)):
            for c in range(nh // CH):
                cps.append(pltpu.async_copy(
                    table_h.at[idx_ref.at[half,
                                          pl.ds(idx_off + c * CH, CH)]],
                    buf.at[pl.ds(c * CH, CH)], sem))
        for cp in cps:
            cp.wait()
        pltpu.sync_copy(rows_e.at[pl.ds(0, nh), pl.ds(0, D)],
                        dst.at[:, pl.ds(0, D)])
        pltpu.sync_copy(rows_o.at[pl.ds(0, nh), pl.ds(0, D)],
                        dst.at[:, pl.ds(D, D)])
    return group


def _sc_gather_ctx(xidx_eo, nidx_eo, xemb, B, K, D):
    BK = B * K
    Dp = 2 * D
    hpw = B // 2 // NW
    nhpw = BK // 2 // NW
    GH = GROUP // 2
    mesh = plsc.VectorSubcoreMesh(core_axis_name="c", subcore_axis_name="s")

    @functools.partial(
        pl.kernel,
        out_type=(
            jax.ShapeDtypeStruct((B // 2, 2 * D), jnp.float32),
            jax.ShapeDtypeStruct((K, B // 2, 2 * D), jnp.float32),
        ),
        mesh=mesh,
        compiler_params=pltpu.CompilerParams(use_tc_tiling_on_sc=False),
        scratch_types=[
            pltpu.VMEM((2, hpw), jnp.int32),
            pltpu.VMEM((2, nhpw), jnp.int32),
            pltpu.VMEM((GH, Dp), jnp.float32),
            pltpu.VMEM((GH, Dp), jnp.float32),
            pltpu.SemaphoreType.DMA,
        ],
    )
    def gather_kernel(xidx_h, nidx_h, xemb_h, ct_pk_o, ne_pk_o,
                      idx_x, idx_n, rows_e, rows_o, sem):
        wid = lax.axis_index("s") * NC + lax.axis_index("c")
        pltpu.sync_copy(xidx_h.at[:, pl.ds(wid * hpw, hpw)], idx_x)
        pltpu.sync_copy(nidx_h.at[:, pl.ds(wid * nhpw, nhpw)], idx_n)
        group = _make_group(rows_e, rows_o, sem, D)
        group(xemb_h, idx_x, 0,
              ct_pk_o.at[pl.ds(wid * hpw, hpw), :], hpw)
        for g in range(nhpw // GH):
            half_row = wid * nhpw + g * GH
            k = half_row // (B // 2)
            j0 = half_row % (B // 2)
            group(xemb_h, idx_n, g * GH,
                  ne_pk_o.at[k].at[pl.ds(j0, GH), :], GH)

    return gather_kernel(xidx_eo, nidx_eo, xemb)


def _sc_gather_ctr(cidx_eo, cemb, B, D):
    Dp = 2 * D
    hpw = B // 2 // NW
    GH = GROUP // 2
    mesh = plsc.VectorSubcoreMesh(core_axis_name="c", subcore_axis_name="s")

    @functools.partial(
        pl.kernel,
        out_type=jax.ShapeDtypeStruct((B // 2, 2 * D), jnp.float32),
        mesh=mesh,
        compiler_params=pltpu.CompilerParams(use_tc_tiling_on_sc=False),
        scratch_types=[
            pltpu.VMEM((2, hpw), jnp.int32),
            pltpu.VMEM((GH, Dp), jnp.float32),
            pltpu.VMEM((GH, Dp), jnp.float32),
            pltpu.SemaphoreType.DMA,
        ],
    )
    def gather_kernel(cidx_h, cemb_h, ce_pk_o, idx_c, rows_e, rows_o, sem):
        wid = lax.axis_index("s") * NC + lax.axis_index("c")
        pltpu.sync_copy(cidx_h.at[:, pl.ds(wid * hpw, hpw)], idx_c)
        group = _make_group(rows_e, rows_o, sem, D)
        group(cemb_h, idx_c, 0,
              ce_pk_o.at[pl.ds(wid * hpw, hpw), :], hpw)

    return gather_kernel(cidx_eo, cemb)


def _tc_transpose_pad(tableT, V, D):
    BLKV = 32768
    nblk = pl.cdiv(V, BLKV)

    def body(t_ref, out_ref):
        x = t_ref[...]
        xt = jnp.swapaxes(x, 0, 1)
        out_ref[...] = jnp.concatenate([xt, jnp.zeros_like(xt)], axis=1)

    return pl.pallas_call(
        body,
        grid=(nblk,),
        in_specs=[pl.BlockSpec((D, BLKV), lambda i: (0, i))],
        out_specs=pl.BlockSpec((BLKV, 2 * D), lambda i: (i, 0)),
        out_shape=jax.ShapeDtypeStruct((V, 2 * D), jnp.float32),
    )(tableT)


def _softplus(x):
    return jnp.maximum(x, 0.0) + jnp.log1p(jnp.exp(-jnp.abs(x)))


def _half_mask(rows, cols):
    r = lax.broadcasted_iota(jnp.int32, (rows, cols), 0)
    c = lax.broadcasted_iota(jnp.int32, (rows, cols), 1)
    return jnp.where(r // 64 == c, 1.0, 0.0).astype(jnp.float32)


def _tc_mlp_pos(ce_pk, ct_pk, W1d, b1d, W2d, b2d, B, D, H):
    BLK = 512
    npk = B // 2
    nblk = npk // BLK

    def body(ce_ref, ct_ref, w1_ref, b1_ref, w2_ref, b2_ref,
             ce2_ref, pos_ref):
        i = pl.program_id(0)
        h = jnp.dot(ce_ref[...], w1_ref[...],
                    preferred_element_type=jnp.float32) + b1_ref[...]
        h = jnp.maximum(h, 0.0)
        ce2 = jnp.dot(h, w2_ref[...],
                      preferred_element_type=jnp.float32) + b2_ref[...]
        ce2_ref[...] = ce2
        prod = ce2 * ct_ref[...]
        pos = jnp.dot(prod, _half_mask(2 * D, 2),
                      preferred_element_type=jnp.float32)
        part = jnp.sum(_softplus(-pos)) * (1.0 / B)

        @pl.when(i == 0)
        def _():
            pos_ref[0, 0] = part

        @pl.when(i != 0)
        def _():
            pos_ref[0, 0] += part

    return pl.pallas_call(
        body,
        grid=(nblk,),
        in_specs=[
            pl.BlockSpec((BLK, 2 * D), lambda i: (i, 0)),
            pl.BlockSpec((BLK, 2 * D), lambda i: (i, 0)),
            pl.BlockSpec((2 * D, 2 * H), lambda i: (0, 0)),
            pl.BlockSpec((1, 2 * H), lambda i: (0, 0)),
            pl.BlockSpec((2 * H, 2 * D), lambda i: (0, 0)),
            pl.BlockSpec((1, 2 * D), lambda i: (0, 0)),
        ],
        out_specs=[
            pl.BlockSpec((BLK, 2 * D), lambda i: (i, 0)),
            pl.BlockSpec(memory_space=pltpu.SMEM),
        ],
        out_shape=[
            jax.ShapeDtypeStruct((npk, 2 * D), jnp.float32),
            jax.ShapeDtypeStruct((1, 1), jnp.float32),
        ],
    )(ce_pk, ct_pk, W1d, b1d, W2d, b2d)


def _tc_neg(neg_pk, ce2_pk, B, K, D):
    BLK = 512
    npk = B // 2
    nblk = npk // BLK

    def body(ne_ref, ce2_ref, out_ref):
        i = pl.program_id(0)
        ce2 = ce2_ref[...]
        prods = [ne_ref[k] * ce2 for k in range(K)]
        pall = jnp.concatenate(prods, axis=1)
        scores = jnp.dot(pall, _half_mask(K * 2 * D, 2 * K),
                         preferred_element_type=jnp.float32)
        part = jnp.sum(_softplus(scores)) * (1.0 / (B * K))

        @pl.when(i == 0)
        def _():
            out_ref[0, 0] = part

        @pl.when(i != 0)
        def _():
            out_ref[0, 0] += part

    return pl.pallas_call(
        body,
        grid=(nblk,),
        in_specs=[
            pl.BlockSpec((K, BLK, 2 * D), lambda i: (0, i, 0)),
            pl.BlockSpec((BLK, 2 * D), lambda i: (i, 0)),
        ],
        out_specs=pl.BlockSpec(memory_space=pltpu.SMEM),
        out_shape=jax.ShapeDtypeStruct((1, 1), jnp.float32),
    )(neg_pk, ce2_pk)


def _blockdiag2(W):
    n, m = W.shape
    z = jnp.zeros((n, m), jnp.float32)
    return jnp.concatenate([
        jnp.concatenate([W, z], axis=1),
        jnp.concatenate([z, W], axis=1),
    ], axis=0)


def kernel(center_word_indices, context_word_indices, negative_word_indices,
           center_emb, context_emb, W1, b1, W2, b2):
    B, K = negative_word_indices.shape
    V, D = center_emb.shape
    H = W1.shape[1]
    cidx = center_word_indices.astype(jnp.int32)
    xidx = context_word_indices.astype(jnp.int32)
    nidx2 = negative_word_indices.astype(jnp.int32).T

    def halves(a):
        return jnp.stack([a[: B // 2], a[B // 2:]])

    nidx_eo = jnp.stack([nidx2[:, : B // 2].reshape(-1),
                         nidx2[:, B // 2:].reshape(-1)])

    xemb_p = _tc_transpose_pad(context_emb.T, V, D)
    ct_pk, neg_pk = _sc_gather_ctx(halves(xidx), nidx_eo, xemb_p, B, K, D)
    cemb_p = _tc_transpose_pad(center_emb.T, V, D)
    ce_pk = _sc_gather_ctr(halves(cidx), cemb_p, B, D)
    W1d = _blockdiag2(W1)
    W2d = _blockdiag2(W2)
    b1d = jnp.concatenate([b1, b1]).reshape(1, 2 * H)
    b2d = jnp.concatenate([b2, b2]).reshape(1, 2 * D)
    ce2_pk, pos_loss = _tc_mlp_pos(ce_pk, ct_pk, W1d, b1d, W2d, b2d, B, D, H)
    neg_loss = _tc_neg(neg_pk, ce2_pk, B, K, D)
    return pos_loss[0, 0] + neg_loss[0, 0]
